# baseline (speedup 1.0000x reference)
import functools

import jax
import jax.numpy as jnp
import numpy as np
from jax import lax
from jax.experimental import pallas as pl
from jax.experimental.pallas import tpu as pltpu
from jax.experimental.pallas import tpu_sc as plsc

_N = 10000
_E = 320000
_D = 128
_DA = 8
_DE = 16
_FCH = 8
_LOG2 = float(np.log(2.0))
_INV_SQRT_D = float(1.0 / np.sqrt(float(_D)))
_INV_SQRT_DE = float(1.0 / np.sqrt(float(_DE)))
_INV_SQRT_FCH = float(1.0 / np.sqrt(float(_FCH)))
_INV_SQRT_AVG = float(1.0 / np.sqrt(32.0))
_INV_SQRT_DDA = float(1.0 / np.sqrt(float(_D * _DA)))

_BN = 2000
_BE = 2000

_NC = 2
_NS = 16
_NW = _NC * _NS
_K = 80
_EPW = _E // _NW
_NCH = _EPW // _K
_NPAD = 10240
_RPT = _NPAD // _NS


def _ssp(v):
    return jnp.maximum(v, 0.0) + jnp.log1p(jnp.exp(-jnp.abs(v))) - _LOG2


def _x_body(nf_ref, w1_ref, x_ref):
    x_ref[...] = jnp.dot(nf_ref[...], w1_ref[...],
                         preferred_element_type=jnp.float32) * _INV_SQRT_D


def _x_call(nf, W1):
    return pl.pallas_call(
        _x_body,
        grid=(_N // _BN,),
        in_specs=[
            pl.BlockSpec((_BN, _D), lambda i: (i, 0)),
            pl.BlockSpec((_D, _D), lambda i: (0, 0)),
        ],
        out_specs=pl.BlockSpec((_BN, _D), lambda i: (i, 0)),
        out_shape=jax.ShapeDtypeStruct((_N, _D), jnp.float32),
    )(nf, W1)


_BEW = 1000
_NWR = _E // 8


def _w_body(eew_ref, wr1bd_ref, wr2bd_ref, out_ref):
    h = jnp.dot(eew_ref[...], wr1bd_ref[...],
                preferred_element_type=jnp.float32) * _INV_SQRT_DE
    h = _ssp(h)
    for j in range(8):
        out_ref[:, j, :] = jnp.dot(
            h, wr2bd_ref[:, 128 * j:128 * (j + 1)],
            preferred_element_type=jnp.float32,
        ) * (_INV_SQRT_FCH * _INV_SQRT_AVG)


def _w_call(eew, Wr1bd, Wr2bd):
    return pl.pallas_call(
        _w_body,
        grid=(_NWR // _BEW,),
        in_specs=[
            pl.BlockSpec((_BEW, 128), lambda i: (i, 0)),
            pl.BlockSpec((128, 64), lambda i: (0, 0)),
            pl.BlockSpec((64, 1024), lambda i: (0, 0)),
        ],
        out_specs=pl.BlockSpec((_BEW, 8, _D), lambda i: (i, 0, 0)),
        out_shape=jax.ShapeDtypeStruct((_NWR, 8, _D), jnp.float32),
    )(eew, Wr1bd, Wr2bd)


def _sc_body(x_hbm, wp_hbm, ea_hbm, src_hbm, dst_hbm, out_hbm,
             src0, src1, src2, src3, dst0, dst1, ea0, ea1,
             rows0, rows1, w0, w1, agg_sh,
             sems0, sems1, sems2, sems3, semd0, semd1, sema0, sema1,
             semg0, semg1, semw0, semw1):
    cid = lax.axis_index("c")
    sid = lax.axis_index("s")
    wid = sid * _NC + cid
    srcv = (src0, src1, src2, src3)
    dstv = (dst0, dst1)
    eav = (ea0, ea1)
    rows = (rows0, rows1)
    wv = (w0, w1)
    sems = (sems0, sems1, sems2, sems3)
    semd = (semd0, semd1)
    sema = (sema0, sema1)
    semg = (semg0, semg1)
    semw = (semw0, semw1)
    base = wid * _EPW

    def _zb(k, carry):
        for j in range(_D // 16):
            rows0[k, pl.ds(j * 16, 16)] = jnp.zeros((16,), jnp.float32)
        return carry
    lax.fori_loop(0, _K, _zb, 0)
    for c in range(_RPT // _K):
        pltpu.sync_copy(rows0, agg_sh.at[pl.ds(sid * _RPT + c * _K, _K)])
    plsc.subcore_barrier()

    def _lsrc(t, c):
        pltpu.async_copy(src_hbm.at[pl.ds(base + t * _K, _K)],
                         srcv[c], sems[c])

    baserow = wid * (_EPW // 8)

    def _issue(t, b, c, wait_src):
        if wait_src:
            pltpu.make_async_copy(src_hbm.at[pl.ds(base + t * _K, _K)],
                                  srcv[c], sems[c]).wait()
        pltpu.async_copy(dst_hbm.at[pl.ds(base + t * _K, _K)],
                         dstv[b], semd[b])
        pltpu.async_copy(ea_hbm.at[pl.ds(base + t * _K, _K)],
                         eav[b].at[pl.ds(0, _K)], sema[b])
        pltpu.async_copy(x_hbm.at[srcv[c]], rows[b], semg[b])
        pltpu.async_copy(wp_hbm.at[pl.ds(baserow + t * (_K // 8), _K // 8)],
                         wv[b], semw[b])

    def _consume(t, b, c):
        pltpu.make_async_copy(x_hbm.at[srcv[c]], rows[b], semg[b]).wait()
        pltpu.make_async_copy(wp_hbm.at[pl.ds(baserow + t * (_K // 8), _K // 8)],
                              wv[b], semw[b]).wait()
        pltpu.make_async_copy(ea_hbm.at[pl.ds(base + t * _K, _K)],
                              eav[b].at[pl.ds(0, _K)], sema[b]).wait()

        def _mul(k8, c2):
            for r in range(8):
                k = k8 * 8 + r
                s = jnp.broadcast_to(eav[b][pl.ds(k, 16)][0], (16,))
                for j in range(_D // 16):
                    sl = pl.ds(j * 16, 16)
                    rows[b][k, sl] = rows[b][k, sl] * (wv[b][k8, r, sl] * s)
            return c2
        lax.fori_loop(0, _K // 8, _mul, 0)
        pltpu.make_async_copy(dst_hbm.at[pl.ds(base + t * _K, _K)],
                              dstv[b], semd[b]).wait()
        pltpu.sync_copy(rows[b], agg_sh.at[dstv[b]], add=True)
        @pl.when(t + 4 < _NCH)
        def _():
            _lsrc(t + 4, c)

    pltpu.sync_copy(src_hbm.at[pl.ds(base, _K)], src0)
    pltpu.sync_copy(src_hbm.at[pl.ds(base + _K, _K)], src1)
    _lsrc(2, 2)
    _lsrc(3, 3)
    _issue(0, 0, 0, False)
    _issue(1, 1, 1, False)

    def _quad(g, carry):
        t0 = g * 4
        _consume(t0, 0, 0)
        _issue(t0 + 2, 0, 2, True)
        _consume(t0 + 1, 1, 1)
        _issue(t0 + 3, 1, 3, True)
        _consume(t0 + 2, 0, 2)

        @pl.when(t0 + 4 < _NCH)
        def _():
            _issue(t0 + 4, 0, 0, True)
        _consume(t0 + 3, 1, 3)

        @pl.when(t0 + 5 < _NCH)
        def _():
            _issue(t0 + 5, 1, 1, True)
        return carry

    lax.fori_loop(0, _NCH // 4, _quad, 0)
    _consume(_NCH - 1, 0, 0)
    plsc.subcore_barrier()
    pltpu.sync_copy(agg_sh.at[pl.ds(sid * _RPT, _RPT)],
                    out_hbm.at[cid, pl.ds(sid * _RPT, _RPT)])


def _sc_call(x, wp, ea1d, src1d, dst1d):
    mesh = plsc.VectorSubcoreMesh(core_axis_name="c", subcore_axis_name="s")
    f = pl.kernel(
        _sc_body,
        out_type=jax.ShapeDtypeStruct((_NC, _NPAD, _D), jnp.float32),
        mesh=mesh,
        scratch_types=[
            pltpu.VMEM((_K,), jnp.int32),
            pltpu.VMEM((_K,), jnp.int32),
            pltpu.VMEM((_K,), jnp.int32),
            pltpu.VMEM((_K,), jnp.int32),
            pltpu.VMEM((_K,), jnp.int32),
            pltpu.VMEM((_K,), jnp.int32),
            pltpu.VMEM((_K + 16,), jnp.float32),
            pltpu.VMEM((_K + 16,), jnp.float32),
            pltpu.VMEM((_K, _D), jnp.float32),
            pltpu.VMEM((_K, _D), jnp.float32),
            pltpu.VMEM((_K // 8, 8, _D), jnp.float32),
            pltpu.VMEM((_K // 8, 8, _D), jnp.float32),
            pltpu.VMEM_SHARED((_NPAD, _D), jnp.float32),
        ] + [pltpu.SemaphoreType.DMA] * 12,
    )
    return f(x, wp, ea1d, src1d, dst1d)


def _final_body(nf_ref, na_ref, p_ref, w2_ref, wsct_ref, out_ref):
    agg = p_ref[0, :, :] + p_ref[1, :, :]
    out_lin = jnp.dot(agg, w2_ref[...],
                      preferred_element_type=jnp.float32) * _INV_SQRT_D
    nf = nf_ref[...]
    na = na_ref[...]
    sc = jnp.zeros_like(out_lin)
    for v in range(_DA):
        sc = sc + jnp.dot(nf * na[:, v:v + 1], wsct_ref[v],
                          preferred_element_type=jnp.float32)
    conv = out_lin + sc * _INV_SQRT_DDA
    out_ref[...] = nf + _ssp(conv)


def _final_call(nf, na, partials, W2, Wsc_t):
    return pl.pallas_call(
        _final_body,
        grid=(_N // _BN,),
        in_specs=[
            pl.BlockSpec((_BN, _D), lambda i: (i, 0)),
            pl.BlockSpec((_BN, _DA), lambda i: (i, 0)),
            pl.BlockSpec((_NC, _BN, _D), lambda i: (0, i, 0)),
            pl.BlockSpec((_D, _D), lambda i: (0, 0)),
            pl.BlockSpec((_DA, _D, _D), lambda i: (0, 0, 0)),
        ],
        out_specs=pl.BlockSpec((_BN, _D), lambda i: (i, 0)),
        out_shape=jax.ShapeDtypeStruct((_N, _D), jnp.float32),
    )(nf, na, partials, W2, Wsc_t)


def kernel(node_features, node_attrs, edge_embedding, edge_attrs, edge_index,
           W1, Wr1, Wr2, W2, Wsc):
    x = _x_call(node_features, W1)
    eew = edge_embedding.reshape(_NWR, 128)
    eye8 = jnp.eye(8, dtype=jnp.float32)
    wp = _w_call(eew, jnp.kron(eye8, Wr1), jnp.kron(eye8, Wr2))
    ea1d = edge_attrs.reshape(_E)
    partials = _sc_call(x, wp, ea1d, edge_index[0], edge_index[1])
    Wsc_t = jnp.transpose(Wsc, (1, 0, 2))
    return _final_call(node_features, node_attrs, partials, W2, Wsc_t)


# speedup vs baseline: 1.5435x; 1.5435x over previous
import functools

import jax
import jax.numpy as jnp
import numpy as np
from jax import lax
from jax.experimental import pallas as pl
from jax.experimental.pallas import tpu as pltpu
from jax.experimental.pallas import tpu_sc as plsc

_N = 10000
_E = 320000
_D = 128
_DA = 8
_DE = 16
_FCH = 8
_LOG2 = float(np.log(2.0))
_INV_SQRT_D = float(1.0 / np.sqrt(float(_D)))
_INV_SQRT_DE = float(1.0 / np.sqrt(float(_DE)))
_INV_SQRT_FCH = float(1.0 / np.sqrt(float(_FCH)))
_INV_SQRT_AVG = float(1.0 / np.sqrt(32.0))
_INV_SQRT_DDA = float(1.0 / np.sqrt(float(_D * _DA)))

_BN = 2000
_BE = 2000

_NC = 2
_NS = 16
_NW = _NC * _NS
_K = 80
_EPW = _E // _NW
_NCH = _EPW // _K
_NPAD = 10240
_RPT = _NPAD // _NS


def _ssp(v):
    return jnp.maximum(v, 0.0) + jnp.log1p(jnp.exp(-jnp.abs(v))) - _LOG2


def _x_body(nf_ref, w1_ref, x_ref):
    x_ref[...] = jnp.dot(nf_ref[...], w1_ref[...],
                         preferred_element_type=jnp.float32) * _INV_SQRT_D


def _x_call(nf, W1):
    return pl.pallas_call(
        _x_body,
        grid=(_N // _BN,),
        in_specs=[
            pl.BlockSpec((_BN, _D), lambda i: (i, 0)),
            pl.BlockSpec((_D, _D), lambda i: (0, 0)),
        ],
        out_specs=pl.BlockSpec((_BN, _D), lambda i: (i, 0)),
        out_shape=jax.ShapeDtypeStruct((_N, _D), jnp.float32),
    )(nf, W1)


_BEW = 2000
_NWR = _E // 8


def _w_body(eew_ref, wr1bd_ref, wr2bd_ref, out_ref):
    h = jnp.dot(eew_ref[...], wr1bd_ref[...],
                preferred_element_type=jnp.float32) * _INV_SQRT_DE
    h = _ssp(h)
    for j in range(8):
        out_ref[:, j, :] = jnp.dot(
            h, wr2bd_ref[:, 128 * j:128 * (j + 1)],
            preferred_element_type=jnp.float32,
        ) * (_INV_SQRT_FCH * _INV_SQRT_AVG)


def _w_call(eew, Wr1bd, Wr2bd):
    return pl.pallas_call(
        _w_body,
        grid=(_NWR // _BEW,),
        in_specs=[
            pl.BlockSpec((_BEW, 128), lambda i: (i, 0)),
            pl.BlockSpec((128, 64), lambda i: (0, 0)),
            pl.BlockSpec((64, 1024), lambda i: (0, 0)),
        ],
        out_specs=pl.BlockSpec((_BEW, 8, _D), lambda i: (i, 0, 0)),
        out_shape=jax.ShapeDtypeStruct((_NWR, 8, _D), jnp.float32),
    )(eew, Wr1bd, Wr2bd)


def _sc_body(x_hbm, wp_hbm, ea_hbm, src_hbm, dst_hbm, out_hbm,
             src0, src1, src2, src3, dst0, dst1, ea0, ea1,
             rows0, rows1, w0, w1, agg_sh,
             sems0, sems1, sems2, sems3, semd0, semd1, sema0, sema1,
             semg0, semg1, semw0, semw1):
    cid = lax.axis_index("c")
    sid = lax.axis_index("s")
    wid = sid * _NC + cid
    srcv = (src0, src1, src2, src3)
    dstv = (dst0, dst1)
    eav = (ea0, ea1)
    rows = (rows0, rows1)
    wv = (w0, w1)
    sems = (sems0, sems1, sems2, sems3)
    semd = (semd0, semd1)
    sema = (sema0, sema1)
    semg = (semg0, semg1)
    semw = (semw0, semw1)
    base = wid * _EPW

    def _zb(k, carry):
        for j in range(_D // 16):
            rows0[k, pl.ds(j * 16, 16)] = jnp.zeros((16,), jnp.float32)
        return carry
    lax.fori_loop(0, _K, _zb, 0)
    for c in range(_RPT // _K):
        pltpu.sync_copy(rows0, agg_sh.at[pl.ds(sid * _RPT + c * _K, _K)])
    plsc.subcore_barrier()

    def _lsrc(t, c):
        pltpu.async_copy(src_hbm.at[pl.ds(base + t * _K, _K)],
                         srcv[c], sems[c])

    def _issue(t, b, c, wait_src):
        if wait_src:
            pltpu.make_async_copy(src_hbm.at[pl.ds(base + t * _K, _K)],
                                  srcv[c], sems[c]).wait()
        pltpu.async_copy(dst_hbm.at[pl.ds(base + t * _K, _K)],
                         dstv[b], semd[b])
        pltpu.async_copy(ea_hbm.at[pl.ds(base + t * _K, _K)],
                         eav[b].at[pl.ds(0, _K)], sema[b])
        pltpu.async_copy(x_hbm.at[srcv[c]], rows[b], semg[b])
        pltpu.async_copy(wp_hbm.at[pl.ds(base + t * _K, _K)], wv[b], semw[b])

    def _consume(t, b, c):
        pltpu.make_async_copy(x_hbm.at[srcv[c]], rows[b], semg[b]).wait()
        pltpu.make_async_copy(wp_hbm.at[pl.ds(base + t * _K, _K)],
                              wv[b], semw[b]).wait()
        pltpu.make_async_copy(ea_hbm.at[pl.ds(base + t * _K, _K)],
                              eav[b].at[pl.ds(0, _K)], sema[b]).wait()

        def _mul(k, c2):
            s = jnp.broadcast_to(eav[b][pl.ds(k, 16)][0], (16,))
            for j in range(_D // 16):
                sl = pl.ds(j * 16, 16)
                rows[b][k, sl] = rows[b][k, sl] * (wv[b][k, sl] * s)
            return c2
        lax.fori_loop(0, _K, _mul, 0)
        pltpu.make_async_copy(dst_hbm.at[pl.ds(base + t * _K, _K)],
                              dstv[b], semd[b]).wait()
        pltpu.sync_copy(rows[b], agg_sh.at[dstv[b]], add=True)
        @pl.when(t + 4 < _NCH)
        def _():
            _lsrc(t + 4, c)

    pltpu.sync_copy(src_hbm.at[pl.ds(base, _K)], src0)
    pltpu.sync_copy(src_hbm.at[pl.ds(base + _K, _K)], src1)
    _lsrc(2, 2)
    _lsrc(3, 3)
    _issue(0, 0, 0, False)
    _issue(1, 1, 1, False)

    def _quad(g, carry):
        t0 = g * 4
        _consume(t0, 0, 0)
        _issue(t0 + 2, 0, 2, True)
        _consume(t0 + 1, 1, 1)
        _issue(t0 + 3, 1, 3, True)
        _consume(t0 + 2, 0, 2)

        @pl.when(t0 + 4 < _NCH)
        def _():
            _issue(t0 + 4, 0, 0, True)
        _consume(t0 + 3, 1, 3)

        @pl.when(t0 + 5 < _NCH)
        def _():
            _issue(t0 + 5, 1, 1, True)
        return carry

    lax.fori_loop(0, _NCH // 4, _quad, 0)
    _consume(_NCH - 1, 0, 0)
    plsc.subcore_barrier()
    pltpu.sync_copy(agg_sh.at[pl.ds(sid * _RPT, _RPT)],
                    out_hbm.at[cid, pl.ds(sid * _RPT, _RPT)])


def _sc_call(x, wp, ea1d, src1d, dst1d):
    mesh = plsc.VectorSubcoreMesh(core_axis_name="c", subcore_axis_name="s")
    f = pl.kernel(
        _sc_body,
        out_type=jax.ShapeDtypeStruct((_NC, _NPAD, _D), jnp.float32),
        mesh=mesh,
        scratch_types=[
            pltpu.VMEM((_K,), jnp.int32),
            pltpu.VMEM((_K,), jnp.int32),
            pltpu.VMEM((_K,), jnp.int32),
            pltpu.VMEM((_K,), jnp.int32),
            pltpu.VMEM((_K,), jnp.int32),
            pltpu.VMEM((_K,), jnp.int32),
            pltpu.VMEM((_K + 16,), jnp.float32),
            pltpu.VMEM((_K + 16,), jnp.float32),
            pltpu.VMEM((_K, _D), jnp.float32),
            pltpu.VMEM((_K, _D), jnp.float32),
            pltpu.VMEM((_K, _D), jnp.float32),
            pltpu.VMEM((_K, _D), jnp.float32),
            pltpu.VMEM_SHARED((_NPAD, _D), jnp.float32),
        ] + [pltpu.SemaphoreType.DMA] * 12,
    )
    return f(x, wp, ea1d, src1d, dst1d)


def _final_body(nf_ref, na_ref, p_ref, w2_ref, wsct_ref, out_ref):
    agg = p_ref[0, :, :] + p_ref[1, :, :]
    out_lin = jnp.dot(agg, w2_ref[...],
                      preferred_element_type=jnp.float32) * _INV_SQRT_D
    nf = nf_ref[...]
    na = na_ref[...]
    sc = jnp.zeros_like(out_lin)
    for v in range(_DA):
        sc = sc + jnp.dot(nf * na[:, v:v + 1], wsct_ref[v],
                          preferred_element_type=jnp.float32)
    conv = out_lin + sc * _INV_SQRT_DDA
    out_ref[...] = nf + _ssp(conv)


def _final_call(nf, na, partials, W2, Wsc_t):
    return pl.pallas_call(
        _final_body,
        grid=(_N // _BN,),
        in_specs=[
            pl.BlockSpec((_BN, _D), lambda i: (i, 0)),
            pl.BlockSpec((_BN, _DA), lambda i: (i, 0)),
            pl.BlockSpec((_NC, _BN, _D), lambda i: (0, i, 0)),
            pl.BlockSpec((_D, _D), lambda i: (0, 0)),
            pl.BlockSpec((_DA, _D, _D), lambda i: (0, 0, 0)),
        ],
        out_specs=pl.BlockSpec((_BN, _D), lambda i: (i, 0)),
        out_shape=jax.ShapeDtypeStruct((_N, _D), jnp.float32),
    )(nf, na, partials, W2, Wsc_t)


def kernel(node_features, node_attrs, edge_embedding, edge_attrs, edge_index,
           W1, Wr1, Wr2, W2, Wsc):
    x = _x_call(node_features, W1)
    eew = edge_embedding.reshape(_NWR, 128)
    eye8 = jnp.eye(8, dtype=jnp.float32)
    wp = _w_call(eew, jnp.kron(eye8, Wr1), jnp.kron(eye8, Wr2))
    wp = wp.reshape(_E, _D)
    ea1d = edge_attrs.reshape(_E)
    partials = _sc_call(x, wp, ea1d, edge_index[0], edge_index[1])
    Wsc_t = jnp.transpose(Wsc, (1, 0, 2))
    return _final_call(node_features, node_attrs, partials, W2, Wsc_t)
